# baseline (device time: 192856 ns/iter reference)
import jax
import jax.numpy as jnp
from jax import lax
from jax.experimental import pallas as pl
from jax.experimental.pallas import tpu as pltpu

N_DEV = 16
B = 2
SQ = 512
SKV = 512
HQ_PER = 8
DH = 64
DMODEL = 768
DQ_PER = HQ_PER * DH
ROWS = B * SQ
CHUNK = ROWS // N_DEV

_sem_signal = getattr(pl, "semaphore_signal", None) or pltpu.semaphore_signal
_sem_wait = getattr(pl, "semaphore_wait", None) or pltpu.semaphore_wait
_run_scoped = getattr(pl, "run_scoped", None) or pltpu.run_scoped
_CompilerParams = getattr(pltpu, "CompilerParams", None) or pltpu.TPUCompilerParams


def kernel(x, Wq, K_ext, V_ext, Wo):
    my = lax.axis_index("i")
    K = lax.dynamic_slice_in_dim(K_ext, my * HQ_PER, HQ_PER, axis=2)
    V = lax.dynamic_slice_in_dim(V_ext, my * HQ_PER, HQ_PER, axis=2)
    K = K.reshape(B, SKV, DQ_PER)
    V = V.reshape(B, SKV, DQ_PER)

    def body(x_ref, wq_ref, k_ref, v_ref, wo_ref, out_ref,
             acc_ref, q_ref, ctx_ref, send_ref, rs_recv_ref, ag_recv_ref,
             send_sem, rs_sems, ag_sems):
        my_pos = lax.axis_index("i")
        left = jnp.mod(my_pos - 1, N_DEV)
        right = jnp.mod(my_pos + 1, N_DEV)

        barrier = pltpu.get_barrier_semaphore()
        for nbr in (left, right):
            _sem_signal(barrier, inc=1, device_id=(nbr,),
                        device_id_type=pl.DeviceIdType.MESH)
        _sem_wait(barrier, 2)

        xb = x_ref[...].reshape(ROWS, DMODEL).astype(jnp.bfloat16)
        wq = wq_ref[...].astype(jnp.bfloat16)
        q_ref[...] = jnp.dot(xb, wq,
                             preferred_element_type=jnp.float32
                             ).astype(jnp.bfloat16)

        ri = lax.broadcasted_iota(jnp.int32, (SQ, SKV), 0) // 64
        ci = lax.broadcasted_iota(jnp.int32, (SQ, SKV), 1) // 64
        mask = ci <= ri

        for b in range(B):
            for h in range(HQ_PER):
                q = q_ref[b * SQ:(b + 1) * SQ, h * DH:(h + 1) * DH]
                k = k_ref[b, :, h * DH:(h + 1) * DH].astype(jnp.bfloat16)
                s = lax.dot_general(
                    q, k, (((1,), (1,)), ((), ())),
                    preferred_element_type=jnp.float32) * 0.125
                s = jnp.where(mask, s, -1e9)
                m = jnp.max(s, axis=1, keepdims=True)
                w = jnp.exp(s - m)
                w = w / jnp.sum(w, axis=1, keepdims=True)
                v = v_ref[b, :, h * DH:(h + 1) * DH].astype(jnp.bfloat16)
                ctx = lax.dot_general(
                    w.astype(jnp.bfloat16), v, (((1,), (0,)), ((), ())),
                    preferred_element_type=jnp.float32)
                ctx_ref[b * SQ:(b + 1) * SQ,
                        h * DH:(h + 1) * DH] = ctx.astype(jnp.bfloat16)

        wo = wo_ref[...].astype(jnp.bfloat16)
        acc_ref[...] = jnp.dot(ctx_ref[...], wo,
                               preferred_element_type=jnp.float32)

        for s_ in range(N_DEV - 1):
            row0 = jnp.mod(my_pos - s_, N_DEV) * CHUNK
            if s_ > 0:
                upd = (acc_ref[pl.ds(row0, CHUNK), :]
                       + rs_recv_ref[s_ - 1].astype(jnp.float32))
                acc_ref[pl.ds(row0, CHUNK), :] = upd
                send_ref[...] = upd.astype(jnp.bfloat16)
            else:
                send_ref[...] = acc_ref[pl.ds(row0, CHUNK), :
                                        ].astype(jnp.bfloat16)
            rdma = pltpu.make_async_remote_copy(
                src_ref=send_ref,
                dst_ref=rs_recv_ref.at[s_],
                send_sem=send_sem,
                recv_sem=rs_sems.at[s_],
                device_id=(right,),
                device_id_type=pl.DeviceIdType.MESH,
            )
            rdma.start()
            rdma.wait()
        own0 = jnp.mod(my_pos + 1, N_DEV) * CHUNK
        acc_ref[pl.ds(own0, CHUNK), :] = (
            acc_ref[pl.ds(own0, CHUNK), :]
            + rs_recv_ref[N_DEV - 2].astype(jnp.float32))

        for t in range(N_DEV - 1):
            if t == 0:
                send_ref[...] = acc_ref[pl.ds(own0, CHUNK), :
                                        ].astype(jnp.bfloat16)
                src = send_ref
            else:
                src = ag_recv_ref.at[t - 1]
            rdma = pltpu.make_async_remote_copy(
                src_ref=src,
                dst_ref=ag_recv_ref.at[t],
                send_sem=send_sem,
                recv_sem=ag_sems.at[t],
                device_id=(right,),
                device_id_type=pl.DeviceIdType.MESH,
            )
            rdma.start()
            rdma.wait()
            c0 = jnp.mod(my_pos - t, N_DEV) * CHUNK
            acc_ref[pl.ds(c0, CHUNK), :] = ag_recv_ref[t].astype(jnp.float32)

        out_ref[0, :, :] = acc_ref[0:SQ, :]
        out_ref[1, :, :] = acc_ref[SQ:ROWS, :]

        def _exit(exit_sem):
            for nbr in (left, right):
                _sem_signal(exit_sem, inc=1, device_id=(nbr,),
                            device_id_type=pl.DeviceIdType.MESH)
            _sem_wait(exit_sem, 2)
        _run_scoped(_exit, pltpu.SemaphoreType.REGULAR)

    return pl.pallas_call(
        body,
        out_shape=jax.ShapeDtypeStruct((B, SQ, DMODEL), jnp.float32),
        in_specs=[pl.BlockSpec(memory_space=pltpu.VMEM)] * 5,
        out_specs=pl.BlockSpec(memory_space=pltpu.VMEM),
        scratch_shapes=[
            pltpu.VMEM((ROWS, DMODEL), jnp.float32),
            pltpu.VMEM((ROWS, DQ_PER), jnp.bfloat16),
            pltpu.VMEM((ROWS, DQ_PER), jnp.bfloat16),
            pltpu.VMEM((CHUNK, DMODEL), jnp.bfloat16),
            pltpu.VMEM((N_DEV - 1, CHUNK, DMODEL), jnp.bfloat16),
            pltpu.VMEM((N_DEV - 1, CHUNK, DMODEL), jnp.bfloat16),
            pltpu.SemaphoreType.DMA,
            pltpu.SemaphoreType.DMA((N_DEV - 1,)),
            pltpu.SemaphoreType.DMA((N_DEV - 1,)),
        ],
        compiler_params=_CompilerParams(collective_id=0),
    )(x, Wq, K, V, Wo)
